# baseline (speedup 1.0000x reference)
import functools

import jax
import jax.numpy as jnp
from jax import lax
from jax.experimental import pallas as pl
from jax.experimental.pallas import tpu as pltpu
from jax.experimental.pallas import tpu_sc as plsc

N = 10000
E = 320000
D = 128

NC = 2
NS = 16
NW = NC * NS
EDGES_PER_W = E // NW
CHUNK = 96
NCHUNK = 105
EDGES_PAD = NCHUNK * CHUNK
NACC = N + NS
RING = 4
ROWS_PER_TILE = 624
ROWS_TAIL = N - NS * ROWS_PER_TILE

_mesh = plsc.VectorSubcoreMesh(core_axis_name="c", subcore_axis_name="s")


@functools.partial(
    pl.kernel,
    out_type=jax.ShapeDtypeStruct((NC * N, D), jnp.float32),
    mesh=_mesh,
    scratch_types=[
        [pltpu.VMEM((CHUNK,), jnp.int32) for _ in range(RING)],
        [pltpu.VMEM((CHUNK,), jnp.int32) for _ in range(RING)],
        [pltpu.VMEM((CHUNK, D), jnp.float32) for _ in range(RING)],
        pltpu.VMEM_SHARED((NACC, D), jnp.float32),
        [pltpu.SemaphoreType.DMA for _ in range(RING)],
        [pltpu.SemaphoreType.DMA for _ in range(RING)],
        [pltpu.SemaphoreType.DMA for _ in range(RING)],
        [pltpu.SemaphoreType.DMA for _ in range(RING)],
        pltpu.SemaphoreType.DMA,
    ],
)
def _sc_agg(h_hbm, src_hbm, dst_hbm, zeros_hbm, out_hbm,
            sidx, didx, rows, acc, gsems, ssems, isems, dsems, zsem):
    c = lax.axis_index("c")
    s = lax.axis_index("s")
    wid = c * NS + s
    ebase = wid * EDGES_PAD

    def refill_sidx(i, k):
        pltpu.async_copy(src_hbm.at[pl.ds(ebase + i * CHUNK, CHUNK)],
                         sidx[k], isems[k])

    def wait_sidx(k):
        pltpu.make_async_copy(src_hbm.at[pl.ds(0, CHUNK)], sidx[k],
                              isems[k]).wait()

    def refill_didx(i, k):
        pltpu.async_copy(dst_hbm.at[pl.ds(ebase + i * CHUNK, CHUNK)],
                         didx[k], dsems[k])

    def wait_didx(k):
        pltpu.make_async_copy(dst_hbm.at[pl.ds(0, CHUNK)], didx[k],
                              dsems[k]).wait()

    def issue_gather(i, k):
        pltpu.async_copy(h_hbm.at[sidx[k]], rows[k], gsems[k])

    def wait_gather(k):
        pltpu.make_async_copy(h_hbm.at[sidx[k]], rows[k], gsems[k]).wait()

    def issue_scatter(k):
        pltpu.async_copy(rows[k], acc.at[didx[k]], ssems[k], add=True)

    def wait_scatter(k):
        pltpu.make_async_copy(rows[k], acc.at[didx[k]], ssems[k]).wait()

    r0 = s * ROWS_PER_TILE
    pltpu.async_copy(zeros_hbm.at[pl.ds(r0, ROWS_PER_TILE)],
                     acc.at[pl.ds(r0, ROWS_PER_TILE)], zsem)

    @pl.when(s == 0)
    def _zero_tail():
        pltpu.async_copy(zeros_hbm.at[pl.ds(NS * ROWS_PER_TILE, ROWS_TAIL)],
                         acc.at[pl.ds(NS * ROWS_PER_TILE, ROWS_TAIL)], zsem)

    refill_sidx(0, 0)
    refill_sidx(1, 1)
    refill_sidx(2, 2)
    refill_didx(0, 0)
    refill_didx(1, 1)
    wait_sidx(0)
    issue_gather(0, 0)
    wait_sidx(1)
    issue_gather(1, 1)

    pltpu.make_async_copy(zeros_hbm.at[pl.ds(0, ROWS_PER_TILE)],
                          acc.at[pl.ds(0, ROWS_PER_TILE)], zsem).wait()

    @pl.when(s == 0)
    def _zero_tail_wait():
        pltpu.make_async_copy(zeros_hbm.at[pl.ds(0, ROWS_TAIL)],
                              acc.at[pl.ds(0, ROWS_TAIL)], zsem).wait()

    plsc.subcore_barrier()

    def step(i, k):
        k2 = (k + 2) % RING
        k3 = (k + 3) % RING

        @pl.when(i >= 2)
        def _drain():
            wait_scatter(k2)

        @pl.when(i + 2 < NCHUNK)
        def _prep():
            refill_didx(i + 2, k2)
            wait_sidx(k2)
            issue_gather(i + 2, k2)

        @pl.when(i + 3 < NCHUNK)
        def _prefetch_sidx():
            refill_sidx(i + 3, k3)

        wait_gather(k)
        wait_didx(k)
        issue_scatter(k)

    step(0, 0)
    step(1, 1)

    def loop_body(j, carry):
        i = 2 + 4 * j
        step(i + 0, 2)
        step(i + 1, 3)
        step(i + 2, 0)
        step(i + 3, 1)
        return carry

    lax.fori_loop(0, (NCHUNK - 5) // 4, loop_body, 0)
    step(NCHUNK - 3, (NCHUNK - 3) % RING)
    step(NCHUNK - 2, (NCHUNK - 2) % RING)
    step(NCHUNK - 1, (NCHUNK - 1) % RING)
    wait_scatter((NCHUNK - 2) % RING)
    wait_scatter((NCHUNK - 1) % RING)

    plsc.subcore_barrier()

    pltpu.sync_copy(acc.at[pl.ds(r0, ROWS_PER_TILE)],
                    out_hbm.at[pl.ds(c * N + r0, ROWS_PER_TILE)])

    @pl.when(s == 0)
    def _out_tail():
        pltpu.sync_copy(acc.at[pl.ds(NS * ROWS_PER_TILE, ROWS_TAIL)],
                        out_hbm.at[pl.ds(c * N + NS * ROWS_PER_TILE,
                                         ROWS_TAIL)])


def _mlp_body(x_ref, p0_ref, p1_ref, w1_ref, b1_ref, w2_ref, b2_ref, o_ref):
    h = x_ref[...] + p0_ref[...] + p1_ref[...]
    t = jnp.dot(h, w1_ref[...], preferred_element_type=jnp.float32)
    t = jnp.maximum(t + b1_ref[...], 0.0)
    o_ref[...] = (jnp.dot(t, w2_ref[...], preferred_element_type=jnp.float32)
                  + b2_ref[...])


BM = 1000


def _tc_mlp(h, parts, W1, b1, W2, b2):
    nblk = N // BM
    return pl.pallas_call(
        _mlp_body,
        grid=(nblk,),
        in_specs=[
            pl.BlockSpec((BM, D), lambda i: (i, 0)),
            pl.BlockSpec((BM, D), lambda i: (i, 0)),
            pl.BlockSpec((BM, D), lambda i: (i + N // BM, 0)),
            pl.BlockSpec((D, D), lambda i: (0, 0)),
            pl.BlockSpec((1, D), lambda i: (0, 0)),
            pl.BlockSpec((D, D), lambda i: (0, 0)),
            pl.BlockSpec((1, D), lambda i: (0, 0)),
        ],
        out_specs=pl.BlockSpec((BM, D), lambda i: (i, 0)),
        out_shape=jax.ShapeDtypeStruct((N, D), jnp.float32),
    )(h, parts, parts, W1, b1, W2, b2)


def kernel(x, edge_index, batch, W1_0, b1_0, W2_0, b2_0,
           W1_1, b1_1, W2_1, b2_1, W1_2, b1_2, W2_2, b2_2):
    pad = EDGES_PAD - EDGES_PER_W
    src = jnp.pad(edge_index[0].reshape(NW, EDGES_PER_W),
                  ((0, 0), (0, pad))).reshape(-1)
    sinks = N + (jnp.arange(NW, dtype=jnp.int32) % NS)
    dst = jnp.concatenate(
        [edge_index[1].reshape(NW, EDGES_PER_W),
         jnp.broadcast_to(sinks[:, None], (NW, pad))], axis=1).reshape(-1)
    zeros = jnp.zeros((N, D), jnp.float32)
    params = [(W1_0, b1_0, W2_0, b2_0),
              (W1_1, b1_1, W2_1, b2_1),
              (W1_2, b1_2, W2_2, b2_2)]
    h = x
    for (W1, b1, W2, b2) in params:
        parts = _sc_agg(h, src, dst, zeros)
        h = _tc_mlp(h, parts, W1, b1.reshape(1, D), W2, b2.reshape(1, D))
    return h


# speedup vs baseline: 1.9517x; 1.9517x over previous
import functools

import jax
import jax.numpy as jnp
from jax import lax
from jax.experimental import pallas as pl
from jax.experimental.pallas import tpu as pltpu
from jax.experimental.pallas import tpu_sc as plsc

N = 10000
E = 320000
D = 128

NC = 2
NS = 16
NW = NC * NS
EDGES_PER_W = E // NW
CHUNK = 80
NCHUNK = EDGES_PER_W // CHUNK
RING = 4
ROWS_PER_TILE = 624
ROWS_TAIL = N - NS * ROWS_PER_TILE

_mesh = plsc.VectorSubcoreMesh(core_axis_name="c", subcore_axis_name="s")


@functools.partial(
    pl.kernel,
    out_type=jax.ShapeDtypeStruct((NC * N, D), jnp.float32),
    mesh=_mesh,
    scratch_types=[
        [pltpu.VMEM((CHUNK,), jnp.int32) for _ in range(RING)],
        [pltpu.VMEM((CHUNK,), jnp.int32) for _ in range(RING)],
        [pltpu.VMEM((CHUNK, D), jnp.float32) for _ in range(RING)],
        pltpu.VMEM_SHARED((N, D), jnp.float32),
        [pltpu.SemaphoreType.DMA for _ in range(RING)],
        [pltpu.SemaphoreType.DMA for _ in range(RING)],
        [pltpu.SemaphoreType.DMA for _ in range(RING)],
        [pltpu.SemaphoreType.DMA for _ in range(RING)],
        pltpu.SemaphoreType.DMA,
    ],
)
def _sc_agg(h_hbm, edges_hbm, zeros_hbm, out_hbm,
            sidx, didx, rows, acc, gsems, ssems, isems, dsems, zsem):
    c = lax.axis_index("c")
    s = lax.axis_index("s")
    wid = c * NS + s
    ebase = wid * EDGES_PER_W

    def refill_sidx(i, k):
        pltpu.async_copy(edges_hbm.at[pl.ds(ebase + i * CHUNK, CHUNK)],
                         sidx[k], isems[k])

    def wait_sidx(k):
        pltpu.make_async_copy(edges_hbm.at[pl.ds(0, CHUNK)], sidx[k],
                              isems[k]).wait()

    def refill_didx(i, k):
        pltpu.async_copy(edges_hbm.at[pl.ds(E + ebase + i * CHUNK, CHUNK)],
                         didx[k], dsems[k])

    def wait_didx(k):
        pltpu.make_async_copy(edges_hbm.at[pl.ds(0, CHUNK)], didx[k],
                              dsems[k]).wait()

    def issue_gather(i, k):
        pltpu.async_copy(h_hbm.at[sidx[k]], rows[k], gsems[k])

    def wait_gather(k):
        pltpu.make_async_copy(h_hbm.at[sidx[k]], rows[k], gsems[k]).wait()

    def issue_scatter(k):
        pltpu.async_copy(rows[k], acc.at[didx[k]], ssems[k], add=True)

    def wait_scatter(k):
        pltpu.make_async_copy(rows[k], acc.at[didx[k]], ssems[k]).wait()

    r0 = s * ROWS_PER_TILE
    pltpu.async_copy(zeros_hbm.at[pl.ds(r0, ROWS_PER_TILE)],
                     acc.at[pl.ds(r0, ROWS_PER_TILE)], zsem)

    @pl.when(s == 0)
    def _zero_tail():
        pltpu.async_copy(zeros_hbm.at[pl.ds(NS * ROWS_PER_TILE, ROWS_TAIL)],
                         acc.at[pl.ds(NS * ROWS_PER_TILE, ROWS_TAIL)], zsem)

    refill_sidx(0, 0)
    refill_sidx(1, 1)
    refill_sidx(2, 2)
    refill_didx(0, 0)
    refill_didx(1, 1)
    wait_sidx(0)
    issue_gather(0, 0)
    wait_sidx(1)
    issue_gather(1, 1)

    pltpu.make_async_copy(zeros_hbm.at[pl.ds(0, ROWS_PER_TILE)],
                          acc.at[pl.ds(0, ROWS_PER_TILE)], zsem).wait()

    @pl.when(s == 0)
    def _zero_tail_wait():
        pltpu.make_async_copy(zeros_hbm.at[pl.ds(0, ROWS_TAIL)],
                              acc.at[pl.ds(0, ROWS_TAIL)], zsem).wait()

    plsc.subcore_barrier()

    def step(i, k):
        k2 = (k + 2) % RING
        k3 = (k + 3) % RING

        @pl.when(i >= 2)
        def _drain():
            wait_scatter(k2)

        @pl.when(i + 2 < NCHUNK)
        def _prep():
            refill_didx(i + 2, k2)
            wait_sidx(k2)
            issue_gather(i + 2, k2)

        @pl.when(i + 3 < NCHUNK)
        def _prefetch_sidx():
            refill_sidx(i + 3, k3)

        wait_gather(k)
        wait_didx(k)
        issue_scatter(k)

    step(0, 0)
    step(1, 1)

    def loop_body(j, carry):
        i = 2 + 4 * j
        step(i + 0, 2)
        step(i + 1, 3)
        step(i + 2, 0)
        step(i + 3, 1)
        return carry

    lax.fori_loop(0, (NCHUNK - 5) // 4, loop_body, 0)
    step(NCHUNK - 3, (NCHUNK - 3) % RING)
    step(NCHUNK - 2, (NCHUNK - 2) % RING)
    step(NCHUNK - 1, (NCHUNK - 1) % RING)
    wait_scatter((NCHUNK - 2) % RING)
    wait_scatter((NCHUNK - 1) % RING)

    plsc.subcore_barrier()

    pltpu.sync_copy(acc.at[pl.ds(r0, ROWS_PER_TILE)],
                    out_hbm.at[pl.ds(c * N + r0, ROWS_PER_TILE)])

    @pl.when(s == 0)
    def _out_tail():
        pltpu.sync_copy(acc.at[pl.ds(NS * ROWS_PER_TILE, ROWS_TAIL)],
                        out_hbm.at[pl.ds(c * N + NS * ROWS_PER_TILE,
                                         ROWS_TAIL)])


def _mlp_body(x_ref, p0_ref, p1_ref, w1_ref, b1_ref, w2_ref, b2_ref, o_ref):
    h = x_ref[...] + p0_ref[...] + p1_ref[...]
    t = jnp.dot(h, w1_ref[...], preferred_element_type=jnp.float32)
    t = jnp.maximum(t + b1_ref[...], 0.0)
    o_ref[...] = (jnp.dot(t, w2_ref[...], preferred_element_type=jnp.float32)
                  + b2_ref[...])


BM = 2000


def _tc_mlp(h, parts, W1, b1, W2, b2):
    nblk = N // BM
    return pl.pallas_call(
        _mlp_body,
        grid=(nblk,),
        in_specs=[
            pl.BlockSpec((BM, D), lambda i: (i, 0)),
            pl.BlockSpec((BM, D), lambda i: (i, 0)),
            pl.BlockSpec((BM, D), lambda i: (i + N // BM, 0)),
            pl.BlockSpec((D, D), lambda i: (0, 0)),
            pl.BlockSpec((1, D), lambda i: (0, 0)),
            pl.BlockSpec((D, D), lambda i: (0, 0)),
            pl.BlockSpec((1, D), lambda i: (0, 0)),
        ],
        out_specs=pl.BlockSpec((BM, D), lambda i: (i, 0)),
        out_shape=jax.ShapeDtypeStruct((N, D), jnp.float32),
    )(h, parts, parts, W1, b1, W2, b2)


def kernel(x, edge_index, batch, W1_0, b1_0, W2_0, b2_0,
           W1_1, b1_1, W2_1, b2_1, W1_2, b1_2, W2_2, b2_2):
    edges = edge_index.reshape(2 * E)
    zeros = jnp.zeros((N, D), jnp.float32)
    params = [(W1_0, b1_0, W2_0, b2_0),
              (W1_1, b1_1, W2_1, b2_1),
              (W1_2, b1_2, W2_2, b2_2)]
    h = x
    for (W1, b1, W2, b2) in params:
        parts = _sc_agg(h, edges, zeros)
        h = _tc_mlp(h, parts, W1, b1.reshape(1, D), W2, b2.reshape(1, D))
    return h


# speedup vs baseline: 2.0185x; 1.0342x over previous
import functools

import jax
import jax.numpy as jnp
from jax import lax
from jax.experimental import pallas as pl
from jax.experimental.pallas import tpu as pltpu
from jax.experimental.pallas import tpu_sc as plsc

N = 10000
E = 320000
D = 128

NC = 2
NS = 16
NW = NC * NS
EDGES_PER_W = E // NW
CHUNK = 80
NCHUNK = EDGES_PER_W // CHUNK
RING = 4
ROWS_PER_TILE = 624
ROWS_TAIL = N - NS * ROWS_PER_TILE

_mesh = plsc.VectorSubcoreMesh(core_axis_name="c", subcore_axis_name="s")


@functools.partial(
    pl.kernel,
    out_type=jax.ShapeDtypeStruct((NC * N, D), jnp.float32),
    mesh=_mesh,
    scratch_types=[
        [pltpu.VMEM((CHUNK,), jnp.int32) for _ in range(RING)],
        [pltpu.VMEM((CHUNK,), jnp.int32) for _ in range(RING)],
        [pltpu.VMEM((CHUNK, D), jnp.float32) for _ in range(RING)],
        pltpu.VMEM((16, D), jnp.float32),
        pltpu.VMEM_SHARED((N, D), jnp.float32),
        [pltpu.SemaphoreType.DMA for _ in range(RING)],
        [pltpu.SemaphoreType.DMA for _ in range(RING)],
        [pltpu.SemaphoreType.DMA for _ in range(RING)],
        [pltpu.SemaphoreType.DMA for _ in range(RING)],
        pltpu.SemaphoreType.DMA,
    ],
)
def _sc_agg(h_hbm, edges_hbm, out_hbm,
            sidx, didx, rows, zbuf, acc, gsems, ssems, isems, dsems, zsem):
    c = lax.axis_index("c")
    s = lax.axis_index("s")
    wid = c * NS + s
    ebase = wid * EDGES_PER_W

    def refill_sidx(i, k):
        pltpu.async_copy(edges_hbm.at[pl.ds(ebase + i * CHUNK, CHUNK)],
                         sidx[k], isems[k])

    def wait_sidx(k):
        pltpu.make_async_copy(edges_hbm.at[pl.ds(0, CHUNK)], sidx[k],
                              isems[k]).wait()

    def refill_didx(i, k):
        pltpu.async_copy(edges_hbm.at[pl.ds(E + ebase + i * CHUNK, CHUNK)],
                         didx[k], dsems[k])

    def wait_didx(k):
        pltpu.make_async_copy(edges_hbm.at[pl.ds(0, CHUNK)], didx[k],
                              dsems[k]).wait()

    def issue_gather(i, k):
        pltpu.async_copy(h_hbm.at[sidx[k]], rows[k], gsems[k])

    def wait_gather(k):
        pltpu.make_async_copy(h_hbm.at[sidx[k]], rows[k], gsems[k]).wait()

    def issue_scatter(k):
        pltpu.async_copy(rows[k], acc.at[didx[k]], ssems[k], add=True)

    def wait_scatter(k):
        pltpu.make_async_copy(rows[k], acc.at[didx[k]], ssems[k]).wait()

    r0 = s * ROWS_PER_TILE
    zv = jnp.zeros((16,), jnp.float32)
    for zi in range(16):
        for zj in range(D // 16):
            zbuf[zi, pl.ds(zj * 16, 16)] = zv

    NZ = ROWS_PER_TILE // 16
    for zj in range(NZ):
        pltpu.async_copy(zbuf, acc.at[pl.ds(r0 + 16 * zj, 16)], zsem)

    @pl.when(s == 0)
    def _zero_tail():
        pltpu.async_copy(zbuf, acc.at[pl.ds(NS * ROWS_PER_TILE, ROWS_TAIL)],
                         zsem)

    refill_sidx(0, 0)
    refill_sidx(1, 1)
    refill_sidx(2, 2)
    refill_didx(0, 0)
    refill_didx(1, 1)
    wait_sidx(0)
    issue_gather(0, 0)
    wait_sidx(1)
    issue_gather(1, 1)

    for zj in range(NZ):
        pltpu.make_async_copy(zbuf, acc.at[pl.ds(0, 16)], zsem).wait()

    @pl.when(s == 0)
    def _zero_tail_wait():
        pltpu.make_async_copy(zbuf, acc.at[pl.ds(0, ROWS_TAIL)], zsem).wait()

    plsc.subcore_barrier()

    def step(i, k):
        k2 = (k + 2) % RING
        k3 = (k + 3) % RING

        @pl.when(i >= 2)
        def _drain():
            wait_scatter(k2)

        @pl.when(i + 2 < NCHUNK)
        def _prep():
            refill_didx(i + 2, k2)
            wait_sidx(k2)
            issue_gather(i + 2, k2)

        @pl.when(i + 3 < NCHUNK)
        def _prefetch_sidx():
            refill_sidx(i + 3, k3)

        wait_gather(k)
        wait_didx(k)
        issue_scatter(k)

    step(0, 0)
    step(1, 1)

    def loop_body(j, carry):
        i = 2 + 4 * j
        step(i + 0, 2)
        step(i + 1, 3)
        step(i + 2, 0)
        step(i + 3, 1)
        return carry

    lax.fori_loop(0, (NCHUNK - 5) // 4, loop_body, 0)
    step(NCHUNK - 3, (NCHUNK - 3) % RING)
    step(NCHUNK - 2, (NCHUNK - 2) % RING)
    step(NCHUNK - 1, (NCHUNK - 1) % RING)
    wait_scatter((NCHUNK - 2) % RING)
    wait_scatter((NCHUNK - 1) % RING)

    plsc.subcore_barrier()

    pltpu.sync_copy(acc.at[pl.ds(r0, ROWS_PER_TILE)],
                    out_hbm.at[pl.ds(c * N + r0, ROWS_PER_TILE)])

    @pl.when(s == 0)
    def _out_tail():
        pltpu.sync_copy(acc.at[pl.ds(NS * ROWS_PER_TILE, ROWS_TAIL)],
                        out_hbm.at[pl.ds(c * N + NS * ROWS_PER_TILE,
                                         ROWS_TAIL)])


def _mlp_body(x_ref, p0_ref, p1_ref, w1_ref, b1_ref, w2_ref, b2_ref, o_ref):
    h = x_ref[...] + p0_ref[...] + p1_ref[...]
    t = jnp.dot(h, w1_ref[...], preferred_element_type=jnp.float32)
    t = jnp.maximum(t + b1_ref[...], 0.0)
    o_ref[...] = (jnp.dot(t, w2_ref[...], preferred_element_type=jnp.float32)
                  + b2_ref[...])


BM = 2000


def _tc_mlp(h, parts, W1, b1, W2, b2):
    nblk = N // BM
    return pl.pallas_call(
        _mlp_body,
        grid=(nblk,),
        in_specs=[
            pl.BlockSpec((BM, D), lambda i: (i, 0)),
            pl.BlockSpec((BM, D), lambda i: (i, 0)),
            pl.BlockSpec((BM, D), lambda i: (i + N // BM, 0)),
            pl.BlockSpec((D, D), lambda i: (0, 0)),
            pl.BlockSpec((1, D), lambda i: (0, 0)),
            pl.BlockSpec((D, D), lambda i: (0, 0)),
            pl.BlockSpec((1, D), lambda i: (0, 0)),
        ],
        out_specs=pl.BlockSpec((BM, D), lambda i: (i, 0)),
        out_shape=jax.ShapeDtypeStruct((N, D), jnp.float32),
    )(h, parts, parts, W1, b1, W2, b2)


def kernel(x, edge_index, batch, W1_0, b1_0, W2_0, b2_0,
           W1_1, b1_1, W2_1, b2_1, W1_2, b1_2, W2_2, b2_2):
    edges = edge_index.reshape(2 * E)
    params = [(W1_0, b1_0, W2_0, b2_0),
              (W1_1, b1_1, W2_1, b2_1),
              (W1_2, b1_2, W2_2, b2_2)]
    h = x
    for (W1, b1, W2, b2) in params:
        parts = _sc_agg(h, edges)
        h = _tc_mlp(h, parts, W1, b1.reshape(1, D), W2, b2.reshape(1, D))
    return h
